# baseline (speedup 1.0000x reference)
import jax
import jax.numpy as jnp
from jax import lax
from jax.experimental import pallas as pl
from jax.experimental.pallas import tpu as pltpu
from jax.experimental.pallas import tpu_sc as plsc

TCB = 2048

NUM_CORES = 2
NUM_SUBCORES = 16
NW = NUM_CORES * NUM_SUBCORES

NSEQ = 4096
NPOS = 200
D = 32
NV = 1000000
LANES = 128
NS_BLK = NSEQ // LANES
P = 5
N_CHUNKS = NPOS // P
CHUNK_TOK = P * LANES
GRP_W = LANES * D


def _emb_body(table_hbm, xb_hbm, out_hbm,
              ib0, ib1, rb0, rb1, tb0, tb1,
              isem0, isem1, gsem0, gsem1, osem0, osem1):
    w = lax.axis_index("s") * NUM_CORES + lax.axis_index("c")

    def idx_cp(g, ib, sem):
        return pltpu.make_async_copy(
            xb_hbm.at[pl.ds(g * P, P), w, 0], ib, sem)

    def out_cps(g, tb, sem):
        for k in range(P):
            base = (g * P + k) * (4 * NS_BLK * 1024) + w * 1024
            for e4 in range(4):
                yield pltpu.make_async_copy(
                    tb.at[pl.ds(k * GRP_W + e4 * 1024, 1024)],
                    out_hbm.at[pl.ds(base + e4 * NS_BLK * 1024, 1024)],
                    sem)

    def gather_cp(ib, rb, k, sem):
        return pltpu.make_async_copy(
            table_hbm.at[ib.at[k]], rb.at[pl.ds(k * LANES, LANES)], sem)

    def fire_gathers(ib, rb, sem):
        for k in range(P):
            gather_cp(ib, rb, k, sem).start()

    def drain_gathers(ib, rb, sem):
        for k in range(P):
            gather_cp(ib, rb, k, sem).wait()

    lane = lax.iota(jnp.int32, 16)

    def transpose(rb, tb):
        def per_tok_blk(k, _):
            rows = [k * LANES + s16 * 16 + lane for s16 in range(8)]
            offs = [k * GRP_W + s16 * 16 + lane for s16 in range(8)]
            for e in range(D):
                ce = jnp.bitwise_and(lane + e, D - 1)
                crot = ce * LANES
                for s16 in range(8):
                    vals = plsc.load_gather(rb, [rows[s16], ce])
                    plsc.store_scatter(tb, [crot + offs[s16]], vals)
            return _
        lax.fori_loop(0, P, per_tok_blk, 0)

    idx_cp(0, ib0, isem0).start()
    idx_cp(0, ib0, isem0).wait()
    fire_gathers(ib0, rb0, gsem0)
    idx_cp(1, ib1, isem1).start()

    def step(i, g, ibA, rbA, tbA, isemA, gsemA, osemA,
             ibB, rbB, tbB, isemB, gsemB, more1, more2):
        drain_gathers(ibA, rbA, gsemA)

        @pl.when(more1)
        def _():
            idx_cp(g + 1, ibB, isemB).wait()
            fire_gathers(ibB, rbB, gsemB)

        @pl.when(more2)
        def _():
            idx_cp(g + 2, ibA, isemA).start()

        @pl.when(i > 0)
        def _():
            for c in out_cps(g - 2, tbA, osemA):
                c.wait()

        transpose(rbA, tbA)
        for c in out_cps(g, tbA, osemA):
            c.start()

    def outer(i, carry):
        g0 = 2 * i
        step(i, g0, ib0, rb0, tb0, isem0, gsem0, osem0,
             ib1, rb1, tb1, isem1, gsem1,
             jnp.bool_(True), g0 + 2 <= N_CHUNKS - 1)
        step(i, g0 + 1, ib1, rb1, tb1, isem1, gsem1, osem1,
             ib0, rb0, tb0, isem0, gsem0,
             g0 + 2 <= N_CHUNKS - 1, g0 + 3 <= N_CHUNKS - 1)
        return carry

    lax.fori_loop(0, N_CHUNKS // 2, outer, 0)

    for c in out_cps(N_CHUNKS - 2, tb0, osem0):
        c.wait()
    for c in out_cps(N_CHUNKS - 1, tb1, osem1):
        c.wait()


def _tc_transpose_body(i_ref, o_ref):
    x = i_ref[...]
    eye = jnp.eye(32, dtype=jnp.float32)
    y = jax.lax.dot_general(
        x, eye, (((0,), (0,)), ((), ())),
        preferred_element_type=jnp.float32,
        precision=jax.lax.Precision.HIGHEST)
    z = y.reshape(TCB // 4, 4, 32)
    o_ref[...] = jnp.concatenate([z[:, q, :] for q in range(4)], axis=1)


def _tc_transpose(tt):
    return pl.pallas_call(
        _tc_transpose_body,
        out_shape=jax.ShapeDtypeStruct((NV // 4, 128), jnp.float32),
        grid=((NV + TCB - 1) // TCB,),
        in_specs=[pl.BlockSpec((32, TCB), lambda i: (0, i))],
        out_specs=pl.BlockSpec((TCB // 4, 128), lambda i: (i, 0)),
    )(tt)


@jax.jit
def kernel(x, table):
    xb = (x.astype(jnp.int32)
           .transpose(1, 0, 2)
           .reshape(NPOS, NS_BLK, LANES, 2)
           .transpose(0, 1, 3, 2))

    mesh = plsc.VectorSubcoreMesh(
        core_axis_name="c", subcore_axis_name="s",
        num_cores=NUM_CORES, num_subcores=NUM_SUBCORES,
    )
    run = pl.kernel(
        _emb_body,
        out_type=jax.ShapeDtypeStruct((NSEQ * NPOS * D,), jnp.float32),
        mesh=mesh,
        scratch_types=[
            pltpu.VMEM((P, LANES), jnp.int32),
            pltpu.VMEM((P, LANES), jnp.int32),
            pltpu.VMEM((CHUNK_TOK, D), jnp.float32),
            pltpu.VMEM((CHUNK_TOK, D), jnp.float32),
            pltpu.VMEM((P * GRP_W,), jnp.float32),
            pltpu.VMEM((P * GRP_W,), jnp.float32),
            pltpu.SemaphoreType.DMA,
            pltpu.SemaphoreType.DMA,
            pltpu.SemaphoreType.DMA,
            pltpu.SemaphoreType.DMA,
            pltpu.SemaphoreType.DMA,
            pltpu.SemaphoreType.DMA,
        ],
        compiler_params=pltpu.CompilerParams(
            use_tc_tiling_on_sc=False, needs_layout_passes=False),
    )
    table_lin = _tc_transpose(table.T).reshape(NV, D)
    out_flat = run(table_lin, xb)
    return (out_flat.reshape(NPOS, D // 8, NS_BLK, 8, LANES)
                    .transpose(2, 4, 0, 1, 3)
                    .reshape(NSEQ, NPOS, D))


# speedup vs baseline: 1.5675x; 1.5675x over previous
import jax
import jax.numpy as jnp
from jax import lax
from jax.experimental import pallas as pl
from jax.experimental.pallas import tpu as pltpu
from jax.experimental.pallas import tpu_sc as plsc

TCB = 4096

NUM_CORES = 2
NUM_SUBCORES = 16
NW = NUM_CORES * NUM_SUBCORES

NSEQ = 4096
NPOS = 200
D = 32
NV = 1000000
LANES = 128
NS_BLK = NSEQ // LANES
P = 5
N_CHUNKS = NPOS // P
CHUNK_TOK = P * LANES
GRP_W = LANES * D


def _emb_body(table_hbm, xb_hbm, out_hbm,
              ib0, ib1, rb0, rb1, tb0, tb1,
              isem0, isem1, gsem0, gsem1, osem0, osem1):
    w = lax.axis_index("s") * NUM_CORES + lax.axis_index("c")

    def idx_cp(g, ib, sem):
        return pltpu.make_async_copy(
            xb_hbm.at[pl.ds(g * P, P), w, 0], ib, sem)

    def out_cps(g, tb, sem):
        for k in range(P):
            base = (g * P + k) * (4 * NS_BLK * 1024) + w * 1024
            for e4 in range(4):
                yield pltpu.make_async_copy(
                    tb.at[pl.ds(k * GRP_W + e4 * 1024, 1024)],
                    out_hbm.at[pl.ds(base + e4 * NS_BLK * 1024, 1024)],
                    sem)

    def gather_cp(ib, rb, k, sem):
        return pltpu.make_async_copy(
            table_hbm.at[ib.at[k]], rb.at[pl.ds(k * LANES, LANES)], sem)

    def fire_gathers(ib, rb, sem):
        for k in range(P):
            gather_cp(ib, rb, k, sem).start()

    def drain_gathers(ib, rb, sem):
        for k in range(P):
            gather_cp(ib, rb, k, sem).wait()

    lane = lax.iota(jnp.int32, 16)

    def transpose(rb, tb):
        def per_tok_blk(k, _):
            rows = [k * LANES + s16 * 16 + lane for s16 in range(8)]
            offs = [k * GRP_W + s16 * 16 + lane for s16 in range(8)]
            for e in range(D):
                ce = jnp.bitwise_and(lane + e, D - 1)
                crot = ce * LANES
                for s16 in range(8):
                    vals = plsc.load_gather(rb, [rows[s16], ce])
                    plsc.store_scatter(tb, [crot + offs[s16]], vals)
            return _
        lax.fori_loop(0, P, per_tok_blk, 0)

    idx_cp(0, ib0, isem0).start()
    idx_cp(0, ib0, isem0).wait()
    fire_gathers(ib0, rb0, gsem0)
    idx_cp(1, ib1, isem1).start()

    def step(i, g, ibA, rbA, tbA, isemA, gsemA, osemA,
             ibB, rbB, tbB, isemB, gsemB, more1, more2):
        drain_gathers(ibA, rbA, gsemA)

        @pl.when(more1)
        def _():
            idx_cp(g + 1, ibB, isemB).wait()
            fire_gathers(ibB, rbB, gsemB)

        @pl.when(more2)
        def _():
            idx_cp(g + 2, ibA, isemA).start()

        @pl.when(i > 0)
        def _():
            for c in out_cps(g - 2, tbA, osemA):
                c.wait()

        transpose(rbA, tbA)
        for c in out_cps(g, tbA, osemA):
            c.start()

    def outer(i, carry):
        g0 = 2 * i
        step(i, g0, ib0, rb0, tb0, isem0, gsem0, osem0,
             ib1, rb1, tb1, isem1, gsem1,
             jnp.bool_(True), g0 + 2 <= N_CHUNKS - 1)
        step(i, g0 + 1, ib1, rb1, tb1, isem1, gsem1, osem1,
             ib0, rb0, tb0, isem0, gsem0,
             g0 + 2 <= N_CHUNKS - 1, g0 + 3 <= N_CHUNKS - 1)
        return carry

    lax.fori_loop(0, N_CHUNKS // 2, outer, 0)

    for c in out_cps(N_CHUNKS - 2, tb0, osem0):
        c.wait()
    for c in out_cps(N_CHUNKS - 1, tb1, osem1):
        c.wait()


def _tc_transpose_body(i_ref, o_ref):
    z = i_ref[...].T.reshape(TCB // 4, 4, 32)
    o_ref[...] = jnp.concatenate([z[:, q, :] for q in range(4)], axis=1)


def _tc_transpose(tt):
    return pl.pallas_call(
        _tc_transpose_body,
        out_shape=jax.ShapeDtypeStruct((NV // 4, 128), jnp.float32),
        grid=((NV + TCB - 1) // TCB,),
        in_specs=[pl.BlockSpec((32, TCB), lambda i: (0, i))],
        out_specs=pl.BlockSpec((TCB // 4, 128), lambda i: (i, 0)),
    )(tt)


@jax.jit
def kernel(x, table):
    xb = (x.astype(jnp.int32)
           .transpose(1, 0, 2)
           .reshape(NPOS, NS_BLK, LANES, 2)
           .transpose(0, 1, 3, 2))

    mesh = plsc.VectorSubcoreMesh(
        core_axis_name="c", subcore_axis_name="s",
        num_cores=NUM_CORES, num_subcores=NUM_SUBCORES,
    )
    run = pl.kernel(
        _emb_body,
        out_type=jax.ShapeDtypeStruct((NSEQ * NPOS * D,), jnp.float32),
        mesh=mesh,
        scratch_types=[
            pltpu.VMEM((P, LANES), jnp.int32),
            pltpu.VMEM((P, LANES), jnp.int32),
            pltpu.VMEM((CHUNK_TOK, D), jnp.float32),
            pltpu.VMEM((CHUNK_TOK, D), jnp.float32),
            pltpu.VMEM((P * GRP_W,), jnp.float32),
            pltpu.VMEM((P * GRP_W,), jnp.float32),
            pltpu.SemaphoreType.DMA,
            pltpu.SemaphoreType.DMA,
            pltpu.SemaphoreType.DMA,
            pltpu.SemaphoreType.DMA,
            pltpu.SemaphoreType.DMA,
            pltpu.SemaphoreType.DMA,
        ],
        compiler_params=pltpu.CompilerParams(
            use_tc_tiling_on_sc=False, needs_layout_passes=False),
    )
    table_lin = _tc_transpose(table.T).reshape(NV, D)
    out_flat = run(table_lin, xb)
    return (out_flat.reshape(NPOS, D // 8, NS_BLK, 8, LANES)
                    .transpose(2, 4, 0, 1, 3)
                    .reshape(NSEQ, NPOS, D))


# speedup vs baseline: 1.5884x; 1.0134x over previous
import jax
import jax.numpy as jnp
from jax import lax
from jax.experimental import pallas as pl
from jax.experimental.pallas import tpu as pltpu
from jax.experimental.pallas import tpu_sc as plsc

TCB = 8192

NUM_CORES = 2
NUM_SUBCORES = 16
NW = NUM_CORES * NUM_SUBCORES

NSEQ = 4096
NPOS = 200
D = 32
NV = 1000000
LANES = 128
NS_BLK = NSEQ // LANES
P = 5
N_CHUNKS = NPOS // P
CHUNK_TOK = P * LANES
GRP_W = LANES * D


def _emb_body(table_hbm, xb_hbm, out_hbm,
              ib0, ib1, rb0, rb1, tb0, tb1,
              isem0, isem1, gsem0, gsem1, osem0, osem1):
    w = lax.axis_index("s") * NUM_CORES + lax.axis_index("c")

    def idx_cp(g, ib, sem):
        return pltpu.make_async_copy(
            xb_hbm.at[pl.ds(g * P, P), w, 0], ib, sem)

    def out_cps(g, tb, sem):
        for k in range(P):
            base = (g * P + k) * (4 * NS_BLK * 1024) + w * 1024
            for e4 in range(4):
                yield pltpu.make_async_copy(
                    tb.at[pl.ds(k * GRP_W + e4 * 1024, 1024)],
                    out_hbm.at[pl.ds(base + e4 * NS_BLK * 1024, 1024)],
                    sem)

    def gather_cp(ib, rb, k, sem):
        return pltpu.make_async_copy(
            table_hbm.at[ib.at[k]], rb.at[pl.ds(k * LANES, LANES)], sem)

    def fire_gathers(ib, rb, sem):
        for k in range(P):
            gather_cp(ib, rb, k, sem).start()

    def drain_gathers(ib, rb, sem):
        for k in range(P):
            gather_cp(ib, rb, k, sem).wait()

    lane = lax.iota(jnp.int32, 16)

    def transpose(rb, tb):
        def per_tok_blk(k, _):
            rows = [k * LANES + s16 * 16 + lane for s16 in range(8)]
            offs = [k * GRP_W + s16 * 16 + lane for s16 in range(8)]
            for e in range(D):
                ce = jnp.bitwise_and(lane + e, D - 1)
                crot = ce * LANES
                for s16 in range(8):
                    vals = plsc.load_gather(rb, [rows[s16], ce])
                    plsc.store_scatter(tb, [crot + offs[s16]], vals)
            return _
        lax.fori_loop(0, P, per_tok_blk, 0)

    idx_cp(0, ib0, isem0).start()
    idx_cp(0, ib0, isem0).wait()
    fire_gathers(ib0, rb0, gsem0)
    idx_cp(1, ib1, isem1).start()

    def step(i, g, ibA, rbA, tbA, isemA, gsemA, osemA,
             ibB, rbB, tbB, isemB, gsemB, more1, more2):
        drain_gathers(ibA, rbA, gsemA)

        @pl.when(more1)
        def _():
            idx_cp(g + 1, ibB, isemB).wait()
            fire_gathers(ibB, rbB, gsemB)

        @pl.when(more2)
        def _():
            idx_cp(g + 2, ibA, isemA).start()

        @pl.when(i > 0)
        def _():
            for c in out_cps(g - 2, tbA, osemA):
                c.wait()

        transpose(rbA, tbA)
        for c in out_cps(g, tbA, osemA):
            c.start()

    def outer(i, carry):
        g0 = 2 * i
        step(i, g0, ib0, rb0, tb0, isem0, gsem0, osem0,
             ib1, rb1, tb1, isem1, gsem1,
             jnp.bool_(True), g0 + 2 <= N_CHUNKS - 1)
        step(i, g0 + 1, ib1, rb1, tb1, isem1, gsem1, osem1,
             ib0, rb0, tb0, isem0, gsem0,
             g0 + 2 <= N_CHUNKS - 1, g0 + 3 <= N_CHUNKS - 1)
        return carry

    lax.fori_loop(0, N_CHUNKS // 2, outer, 0)

    for c in out_cps(N_CHUNKS - 2, tb0, osem0):
        c.wait()
    for c in out_cps(N_CHUNKS - 1, tb1, osem1):
        c.wait()


def _tc_transpose_body(i_ref, o_ref):
    z = i_ref[...].T.reshape(TCB // 4, 4, 32)
    o_ref[...] = jnp.concatenate([z[:, q, :] for q in range(4)], axis=1)


def _tc_transpose(tt):
    return pl.pallas_call(
        _tc_transpose_body,
        out_shape=jax.ShapeDtypeStruct((NV // 4, 128), jnp.float32),
        grid=((NV + TCB - 1) // TCB,),
        in_specs=[pl.BlockSpec((32, TCB), lambda i: (0, i))],
        out_specs=pl.BlockSpec((TCB // 4, 128), lambda i: (i, 0)),
    )(tt)


@jax.jit
def kernel(x, table):
    xb = (x.astype(jnp.int32)
           .transpose(1, 0, 2)
           .reshape(NPOS, NS_BLK, LANES, 2)
           .transpose(0, 1, 3, 2))

    mesh = plsc.VectorSubcoreMesh(
        core_axis_name="c", subcore_axis_name="s",
        num_cores=NUM_CORES, num_subcores=NUM_SUBCORES,
    )
    run = pl.kernel(
        _emb_body,
        out_type=jax.ShapeDtypeStruct((NSEQ * NPOS * D,), jnp.float32),
        mesh=mesh,
        scratch_types=[
            pltpu.VMEM((P, LANES), jnp.int32),
            pltpu.VMEM((P, LANES), jnp.int32),
            pltpu.VMEM((CHUNK_TOK, D), jnp.float32),
            pltpu.VMEM((CHUNK_TOK, D), jnp.float32),
            pltpu.VMEM((P * GRP_W,), jnp.float32),
            pltpu.VMEM((P * GRP_W,), jnp.float32),
            pltpu.SemaphoreType.DMA,
            pltpu.SemaphoreType.DMA,
            pltpu.SemaphoreType.DMA,
            pltpu.SemaphoreType.DMA,
            pltpu.SemaphoreType.DMA,
            pltpu.SemaphoreType.DMA,
        ],
        compiler_params=pltpu.CompilerParams(
            use_tc_tiling_on_sc=False, needs_layout_passes=False),
    )
    table_lin = _tc_transpose(table.T).reshape(NV, D)
    out_flat = run(table_lin, xb)
    return (out_flat.reshape(NPOS, D // 8, NS_BLK, 8, LANES)
                    .transpose(2, 4, 0, 1, 3)
                    .reshape(NSEQ, NPOS, D))


# speedup vs baseline: 1.6050x; 1.0104x over previous
import jax
import jax.numpy as jnp
from jax import lax
from jax.experimental import pallas as pl
from jax.experimental.pallas import tpu as pltpu
from jax.experimental.pallas import tpu_sc as plsc

TCB = 16384

NUM_CORES = 2
NUM_SUBCORES = 16
NW = NUM_CORES * NUM_SUBCORES

NSEQ = 4096
NPOS = 200
D = 32
NV = 1000000
LANES = 128
NS_BLK = NSEQ // LANES
P = 5
N_CHUNKS = NPOS // P
CHUNK_TOK = P * LANES
GRP_W = LANES * D


def _emb_body(table_hbm, xb_hbm, out_hbm,
              ib0, ib1, rb0, rb1, tb0, tb1,
              isem0, isem1, gsem0, gsem1, osem0, osem1):
    w = lax.axis_index("s") * NUM_CORES + lax.axis_index("c")

    def idx_cp(g, ib, sem):
        return pltpu.make_async_copy(
            xb_hbm.at[pl.ds(g * P, P), w, 0], ib, sem)

    def out_cps(g, tb, sem):
        for k in range(P):
            base = (g * P + k) * (4 * NS_BLK * 1024) + w * 1024
            for e4 in range(4):
                yield pltpu.make_async_copy(
                    tb.at[pl.ds(k * GRP_W + e4 * 1024, 1024)],
                    out_hbm.at[pl.ds(base + e4 * NS_BLK * 1024, 1024)],
                    sem)

    def gather_cp(ib, rb, k, sem):
        return pltpu.make_async_copy(
            table_hbm.at[ib.at[k]], rb.at[pl.ds(k * LANES, LANES)], sem)

    def fire_gathers(ib, rb, sem):
        for k in range(P):
            gather_cp(ib, rb, k, sem).start()

    def drain_gathers(ib, rb, sem):
        for k in range(P):
            gather_cp(ib, rb, k, sem).wait()

    lane = lax.iota(jnp.int32, 16)

    def transpose(rb, tb):
        def per_tok_blk(k, _):
            rows = [k * LANES + s16 * 16 + lane for s16 in range(8)]
            offs = [k * GRP_W + s16 * 16 + lane for s16 in range(8)]
            for e in range(D):
                ce = jnp.bitwise_and(lane + e, D - 1)
                crot = ce * LANES
                for s16 in range(8):
                    vals = plsc.load_gather(rb, [rows[s16], ce])
                    plsc.store_scatter(tb, [crot + offs[s16]], vals)
            return _
        lax.fori_loop(0, P, per_tok_blk, 0)

    idx_cp(0, ib0, isem0).start()
    idx_cp(0, ib0, isem0).wait()
    fire_gathers(ib0, rb0, gsem0)
    idx_cp(1, ib1, isem1).start()

    def step(i, g, ibA, rbA, tbA, isemA, gsemA, osemA,
             ibB, rbB, tbB, isemB, gsemB, more1, more2):
        drain_gathers(ibA, rbA, gsemA)

        @pl.when(more1)
        def _():
            idx_cp(g + 1, ibB, isemB).wait()
            fire_gathers(ibB, rbB, gsemB)

        @pl.when(more2)
        def _():
            idx_cp(g + 2, ibA, isemA).start()

        @pl.when(i > 0)
        def _():
            for c in out_cps(g - 2, tbA, osemA):
                c.wait()

        transpose(rbA, tbA)
        for c in out_cps(g, tbA, osemA):
            c.start()

    def outer(i, carry):
        g0 = 2 * i
        step(i, g0, ib0, rb0, tb0, isem0, gsem0, osem0,
             ib1, rb1, tb1, isem1, gsem1,
             jnp.bool_(True), g0 + 2 <= N_CHUNKS - 1)
        step(i, g0 + 1, ib1, rb1, tb1, isem1, gsem1, osem1,
             ib0, rb0, tb0, isem0, gsem0,
             g0 + 2 <= N_CHUNKS - 1, g0 + 3 <= N_CHUNKS - 1)
        return carry

    lax.fori_loop(0, N_CHUNKS // 2, outer, 0)

    for c in out_cps(N_CHUNKS - 2, tb0, osem0):
        c.wait()
    for c in out_cps(N_CHUNKS - 1, tb1, osem1):
        c.wait()


def _tc_transpose_body(i_ref, o_ref):
    z = i_ref[...].T.reshape(TCB // 4, 4, 32)
    o_ref[...] = jnp.concatenate([z[:, q, :] for q in range(4)], axis=1)


def _tc_transpose(tt):
    return pl.pallas_call(
        _tc_transpose_body,
        out_shape=jax.ShapeDtypeStruct((NV // 4, 128), jnp.float32),
        grid=((NV + TCB - 1) // TCB,),
        in_specs=[pl.BlockSpec((32, TCB), lambda i: (0, i))],
        out_specs=pl.BlockSpec((TCB // 4, 128), lambda i: (i, 0)),
    )(tt)


@jax.jit
def kernel(x, table):
    xb = (x.astype(jnp.int32)
           .transpose(1, 0, 2)
           .reshape(NPOS, NS_BLK, LANES, 2)
           .transpose(0, 1, 3, 2))

    mesh = plsc.VectorSubcoreMesh(
        core_axis_name="c", subcore_axis_name="s",
        num_cores=NUM_CORES, num_subcores=NUM_SUBCORES,
    )
    run = pl.kernel(
        _emb_body,
        out_type=jax.ShapeDtypeStruct((NSEQ * NPOS * D,), jnp.float32),
        mesh=mesh,
        scratch_types=[
            pltpu.VMEM((P, LANES), jnp.int32),
            pltpu.VMEM((P, LANES), jnp.int32),
            pltpu.VMEM((CHUNK_TOK, D), jnp.float32),
            pltpu.VMEM((CHUNK_TOK, D), jnp.float32),
            pltpu.VMEM((P * GRP_W,), jnp.float32),
            pltpu.VMEM((P * GRP_W,), jnp.float32),
            pltpu.SemaphoreType.DMA,
            pltpu.SemaphoreType.DMA,
            pltpu.SemaphoreType.DMA,
            pltpu.SemaphoreType.DMA,
            pltpu.SemaphoreType.DMA,
            pltpu.SemaphoreType.DMA,
        ],
        compiler_params=pltpu.CompilerParams(
            use_tc_tiling_on_sc=False, needs_layout_passes=False),
    )
    table_lin = _tc_transpose(table.T).reshape(NV, D)
    out_flat = run(table_lin, xb)
    return (out_flat.reshape(NPOS, D // 8, NS_BLK, 8, LANES)
                    .transpose(2, 4, 0, 1, 3)
                    .reshape(NSEQ, NPOS, D))


# speedup vs baseline: 1.6124x; 1.0046x over previous
import jax
import jax.numpy as jnp
from jax import lax
from jax.experimental import pallas as pl
from jax.experimental.pallas import tpu as pltpu
from jax.experimental.pallas import tpu_sc as plsc

TCB = 32768

NUM_CORES = 2
NUM_SUBCORES = 16
NW = NUM_CORES * NUM_SUBCORES

NSEQ = 4096
NPOS = 200
D = 32
NV = 1000000
LANES = 128
NS_BLK = NSEQ // LANES
P = 5
N_CHUNKS = NPOS // P
CHUNK_TOK = P * LANES
GRP_W = LANES * D


def _emb_body(table_hbm, xb_hbm, out_hbm,
              ib0, ib1, rb0, rb1, tb0, tb1,
              isem0, isem1, gsem0, gsem1, osem0, osem1):
    w = lax.axis_index("s") * NUM_CORES + lax.axis_index("c")

    def idx_cp(g, ib, sem):
        return pltpu.make_async_copy(
            xb_hbm.at[pl.ds(g * P, P), w, 0], ib, sem)

    def out_cps(g, tb, sem):
        for k in range(P):
            base = (g * P + k) * (4 * NS_BLK * 1024) + w * 1024
            for e4 in range(4):
                yield pltpu.make_async_copy(
                    tb.at[pl.ds(k * GRP_W + e4 * 1024, 1024)],
                    out_hbm.at[pl.ds(base + e4 * NS_BLK * 1024, 1024)],
                    sem)

    def gather_cp(ib, rb, k, sem):
        return pltpu.make_async_copy(
            table_hbm.at[ib.at[k]], rb.at[pl.ds(k * LANES, LANES)], sem)

    def fire_gathers(ib, rb, sem):
        for k in range(P):
            gather_cp(ib, rb, k, sem).start()

    def drain_gathers(ib, rb, sem):
        for k in range(P):
            gather_cp(ib, rb, k, sem).wait()

    lane = lax.iota(jnp.int32, 16)

    def transpose(rb, tb):
        def per_tok_blk(k, _):
            rows = [k * LANES + s16 * 16 + lane for s16 in range(8)]
            offs = [k * GRP_W + s16 * 16 + lane for s16 in range(8)]
            for e in range(D):
                ce = jnp.bitwise_and(lane + e, D - 1)
                crot = ce * LANES
                for s16 in range(8):
                    vals = plsc.load_gather(rb, [rows[s16], ce])
                    plsc.store_scatter(tb, [crot + offs[s16]], vals)
            return _
        lax.fori_loop(0, P, per_tok_blk, 0)

    idx_cp(0, ib0, isem0).start()
    idx_cp(0, ib0, isem0).wait()
    fire_gathers(ib0, rb0, gsem0)
    idx_cp(1, ib1, isem1).start()

    def step(i, g, ibA, rbA, tbA, isemA, gsemA, osemA,
             ibB, rbB, tbB, isemB, gsemB, more1, more2):
        drain_gathers(ibA, rbA, gsemA)

        @pl.when(more1)
        def _():
            idx_cp(g + 1, ibB, isemB).wait()
            fire_gathers(ibB, rbB, gsemB)

        @pl.when(more2)
        def _():
            idx_cp(g + 2, ibA, isemA).start()

        @pl.when(i > 0)
        def _():
            for c in out_cps(g - 2, tbA, osemA):
                c.wait()

        transpose(rbA, tbA)
        for c in out_cps(g, tbA, osemA):
            c.start()

    def outer(i, carry):
        g0 = 2 * i
        step(i, g0, ib0, rb0, tb0, isem0, gsem0, osem0,
             ib1, rb1, tb1, isem1, gsem1,
             jnp.bool_(True), g0 + 2 <= N_CHUNKS - 1)
        step(i, g0 + 1, ib1, rb1, tb1, isem1, gsem1, osem1,
             ib0, rb0, tb0, isem0, gsem0,
             g0 + 2 <= N_CHUNKS - 1, g0 + 3 <= N_CHUNKS - 1)
        return carry

    lax.fori_loop(0, N_CHUNKS // 2, outer, 0)

    for c in out_cps(N_CHUNKS - 2, tb0, osem0):
        c.wait()
    for c in out_cps(N_CHUNKS - 1, tb1, osem1):
        c.wait()


def _tc_transpose_body(i_ref, o_ref):
    z = i_ref[...].T.reshape(TCB // 4, 4, 32)
    o_ref[...] = jnp.concatenate([z[:, q, :] for q in range(4)], axis=1)


def _tc_transpose(tt):
    return pl.pallas_call(
        _tc_transpose_body,
        out_shape=jax.ShapeDtypeStruct((NV // 4, 128), jnp.float32),
        grid=((NV + TCB - 1) // TCB,),
        in_specs=[pl.BlockSpec((32, TCB), lambda i: (0, i))],
        out_specs=pl.BlockSpec((TCB // 4, 128), lambda i: (i, 0)),
    )(tt)


@jax.jit
def kernel(x, table):
    xb = (x.astype(jnp.int32)
           .transpose(1, 0, 2)
           .reshape(NPOS, NS_BLK, LANES, 2)
           .transpose(0, 1, 3, 2))

    mesh = plsc.VectorSubcoreMesh(
        core_axis_name="c", subcore_axis_name="s",
        num_cores=NUM_CORES, num_subcores=NUM_SUBCORES,
    )
    run = pl.kernel(
        _emb_body,
        out_type=jax.ShapeDtypeStruct((NSEQ * NPOS * D,), jnp.float32),
        mesh=mesh,
        scratch_types=[
            pltpu.VMEM((P, LANES), jnp.int32),
            pltpu.VMEM((P, LANES), jnp.int32),
            pltpu.VMEM((CHUNK_TOK, D), jnp.float32),
            pltpu.VMEM((CHUNK_TOK, D), jnp.float32),
            pltpu.VMEM((P * GRP_W,), jnp.float32),
            pltpu.VMEM((P * GRP_W,), jnp.float32),
            pltpu.SemaphoreType.DMA,
            pltpu.SemaphoreType.DMA,
            pltpu.SemaphoreType.DMA,
            pltpu.SemaphoreType.DMA,
            pltpu.SemaphoreType.DMA,
            pltpu.SemaphoreType.DMA,
        ],
        compiler_params=pltpu.CompilerParams(
            use_tc_tiling_on_sc=False, needs_layout_passes=False),
    )
    table_lin = _tc_transpose(table.T).reshape(NV, D)
    out_flat = run(table_lin, xb)
    return (out_flat.reshape(NPOS, D // 8, NS_BLK, 8, LANES)
                    .transpose(2, 4, 0, 1, 3)
                    .reshape(NSEQ, NPOS, D))
